# baseline (speedup 1.0000x reference)
import functools

import jax
import jax.numpy as jnp
from jax import lax
from jax.experimental import pallas as pl
from jax.experimental.pallas import tpu as pltpu
from jax.experimental.pallas import tpu_sc as plsc

NC = 2
NS = 16
NW = NC * NS
CHUNK = 128
BT = 640


def _gather_body(nrow, emb, tok_hbm, ix_hbm, out_hbm, idx_v, rows_v, sem):
    wid = lax.axis_index("s") * NC + lax.axis_index("c")
    base = wid * nrow
    pltpu.sync_copy(ix_hbm.at[pl.ds(base, nrow)], idx_v)
    copies = [
        pltpu.async_copy(tok_hbm.at[idx_v.at[j]], rows_v.at[j], sem)
        for j in range(nrow)
    ]
    for c in copies:
        c.wait()
    pltpu.sync_copy(rows_v, out_hbm.at[pl.ds(base, nrow)])


def _sc_gather(token_emb, ix_flat):
    n = ix_flat.shape[0]
    emb = token_emb.shape[1]
    nrow = n // (NW * CHUNK)
    ix2 = ix_flat.reshape(NW * nrow, CHUNK)
    mesh = plsc.VectorSubcoreMesh(core_axis_name="c", subcore_axis_name="s")
    f = pl.kernel(
        functools.partial(_gather_body, nrow, emb),
        out_type=jax.ShapeDtypeStruct((NW * nrow, CHUNK, emb), jnp.float32),
        mesh=mesh,
        scratch_types=[
            pltpu.VMEM((nrow, CHUNK), jnp.int32),
            pltpu.VMEM((nrow, CHUNK, emb), jnp.float32),
            pltpu.SemaphoreType.DMA,
        ],
        compiler_params=pltpu.CompilerParams(use_tc_tiling_on_sc=False),
    )
    return f(token_emb, ix2)


def _head_body(x_ref, p_ref, w_ref, b_ref, o_ref):
    x = x_ref[...] + p_ref[...]
    o_ref[...] = (
        jnp.dot(x, w_ref[...], preferred_element_type=jnp.float32) + b_ref[...]
    )


def _tc_head(tok2d, ptile, W, b2d):
    n, emb = tok2d.shape
    vocab = W.shape[1]
    return pl.pallas_call(
        _head_body,
        grid=(n // BT,),
        in_specs=[
            pl.BlockSpec((BT, emb), lambda i: (i, 0)),
            pl.BlockSpec((BT, emb), lambda i: (0, 0)),
            pl.BlockSpec((emb, vocab), lambda i: (0, 0)),
            pl.BlockSpec((1, vocab), lambda i: (0, 0)),
        ],
        out_specs=pl.BlockSpec((BT, vocab), lambda i: (i, 0)),
        out_shape=jax.ShapeDtypeStruct((n, vocab), jnp.float32),
    )(tok2d, ptile, W, b2d)


def kernel(ix, token_emb, pos_emb, W, b):
    bsz, tlen = ix.shape
    n = bsz * tlen
    vocab = W.shape[1]
    emb = token_emb.shape[1]
    tok = _sc_gather(token_emb, ix.reshape(n).astype(jnp.int32))
    ptile = jnp.tile(pos_emb, (BT // tlen, 1))
    logits2d = _tc_head(tok.reshape(n, emb), ptile, W, b.reshape(1, vocab))
    return logits2d.reshape(bsz, tlen, vocab)
